# baseline (speedup 1.0000x reference)
import jax
import jax.numpy as jnp
from jax.experimental import pallas as pl
from jax.experimental.pallas import tpu as pltpu

N_EXP = 64
K = 8


def _gate_body(x_ref, w_ref, idx_ref, wgt_ref, cnt_ref):
    x = x_ref[...]
    w = w_ref[...]
    logits = jax.lax.dot_general(
        x, w, (((1,), (1,)), ((), ())), preferred_element_type=jnp.float32
    )
    rowmax = jnp.max(logits, axis=1, keepdims=True)
    ex = jnp.exp(logits - rowmax)
    scores = ex / jnp.sum(ex, axis=1, keepdims=True)
    col = jax.lax.broadcasted_iota(jnp.int32, scores.shape, 1)
    s = scores
    sel_mask = jnp.zeros(scores.shape, jnp.bool_)
    idxs = []
    for _ in range(K):
        sel = jnp.argmax(s, axis=1).astype(jnp.int32)[:, None]
        idxs.append(sel)
        hit = col == sel
        sel_mask = jnp.logical_or(sel_mask, hit)
        s = jnp.where(hit, -jnp.inf, s)
    topi = jnp.concatenate(idxs, axis=1).astype(jnp.int32)
    topv = jnp.take_along_axis(scores, topi, axis=1)
    wgt = topv / (jnp.sum(topv, axis=1, keepdims=True) + 1e-20)
    idx_ref[...] = topi
    wgt_ref[...] = wgt
    cnt = jnp.sum(sel_mask.astype(jnp.int32), axis=0, keepdims=True)

    @pl.when(pl.program_id(0) == 0)
    def _init():
        cnt_ref[...] = jnp.zeros_like(cnt_ref)

    cnt_ref[...] += cnt


def kernel(hidden_states, weight):
    bsz, seq, d = hidden_states.shape
    tokens = bsz * seq
    x = hidden_states.reshape(tokens, d)
    bt = 512
    grid = (tokens // bt,)
    idx, wgt, cnt = pl.pallas_call(
        _gate_body,
        grid=grid,
        in_specs=[
            pl.BlockSpec((bt, d), lambda i: (i, 0)),
            pl.BlockSpec((N_EXP, d), lambda i: (0, 0)),
        ],
        out_specs=[
            pl.BlockSpec((bt, K), lambda i: (i, 0)),
            pl.BlockSpec((bt, K), lambda i: (i, 0)),
            pl.BlockSpec((1, N_EXP), lambda i: (0, 0)),
        ],
        out_shape=[
            jax.ShapeDtypeStruct((tokens, K), jnp.int32),
            jax.ShapeDtypeStruct((tokens, K), jnp.float32),
            jax.ShapeDtypeStruct((1, N_EXP), jnp.int32),
        ],
    )(x, weight)
    return idx, wgt, cnt.reshape(N_EXP)


# speedup vs baseline: 1.0391x; 1.0391x over previous
import jax
import jax.numpy as jnp
from jax.experimental import pallas as pl
from jax.experimental.pallas import tpu as pltpu

N_EXP = 64
K = 8


def _gate_body(x_ref, w_ref, idx_ref, wgt_ref, cnt_ref):
    x = x_ref[...]
    w = w_ref[...]
    logits = jax.lax.dot_general(
        x, w, (((1,), (1,)), ((), ())), preferred_element_type=jnp.float32
    )
    rowmax = jnp.max(logits, axis=1, keepdims=True)
    ex = jnp.exp(logits - rowmax)
    scores = ex / jnp.sum(ex, axis=1, keepdims=True)
    col = jax.lax.broadcasted_iota(jnp.int32, scores.shape, 1)
    s = scores
    sel_mask = jnp.zeros(scores.shape, jnp.bool_)
    idxs = []
    for _ in range(K):
        sel = jnp.argmax(s, axis=1).astype(jnp.int32)[:, None]
        idxs.append(sel)
        hit = col == sel
        sel_mask = jnp.logical_or(sel_mask, hit)
        s = jnp.where(hit, -jnp.inf, s)
    topi = jnp.concatenate(idxs, axis=1).astype(jnp.int32)
    topv = jnp.take_along_axis(scores, topi, axis=1)
    wgt = topv / (jnp.sum(topv, axis=1, keepdims=True) + 1e-20)
    idx_ref[...] = topi
    wgt_ref[...] = wgt
    cnt = jnp.sum(sel_mask.astype(jnp.int32), axis=0, keepdims=True)

    @pl.when(pl.program_id(0) == 0)
    def _init():
        cnt_ref[...] = jnp.zeros_like(cnt_ref)

    cnt_ref[...] += cnt


def kernel(hidden_states, weight):
    bsz, seq, d = hidden_states.shape
    tokens = bsz * seq
    x = hidden_states.reshape(tokens, d)
    bt = 1024
    grid = (tokens // bt,)
    idx, wgt, cnt = pl.pallas_call(
        _gate_body,
        grid=grid,
        in_specs=[
            pl.BlockSpec((bt, d), lambda i: (i, 0)),
            pl.BlockSpec((N_EXP, d), lambda i: (0, 0)),
        ],
        out_specs=[
            pl.BlockSpec((bt, K), lambda i: (i, 0)),
            pl.BlockSpec((bt, K), lambda i: (i, 0)),
            pl.BlockSpec((1, N_EXP), lambda i: (0, 0)),
        ],
        out_shape=[
            jax.ShapeDtypeStruct((tokens, K), jnp.int32),
            jax.ShapeDtypeStruct((tokens, K), jnp.float32),
            jax.ShapeDtypeStruct((1, N_EXP), jnp.int32),
        ],
    )(x, weight)
    return idx, wgt, cnt.reshape(N_EXP)
